# baseline (speedup 1.0000x reference)
import functools

import jax
import jax.numpy as jnp
from jax import lax
from jax.experimental import pallas as pl
from jax.experimental.pallas import tpu as pltpu
from jax.experimental.pallas import tpu_sc as plsc

BATCH = 16384
SEQ = 50
POOL = 20
VOCAB = 1000000
DIM = 64
WIDE = 128

_INFO = plsc.get_sparse_core_info()
_NC = _INFO.num_cores
_NS = _INFO.num_subcores
_NW = _NC * _NS
_PER_W = BATCH // _NW
_TOK_W = _PER_W * POOL
_C = 8
_G = 2
_NCHUNK = _PER_W // _C
_NGRP = _TOK_W // 80


def _tc_table_w(table, W):
    blk = 8000

    def mm(x_ref, w_ref, o_ref):
        r = lax.dot_general(
            x_ref[...], w_ref[...],
            dimension_numbers=(((1,), (1,)), ((), ())),
            precision=lax.Precision.HIGHEST,
            preferred_element_type=jnp.float32,
        )
        o_ref[...] = jnp.concatenate([r, jnp.zeros_like(r)], axis=1)

    return pl.pallas_call(
        mm,
        grid=(VOCAB // blk,),
        in_specs=[
            pl.BlockSpec((blk, DIM), lambda i: (i, 0)),
            pl.BlockSpec((DIM, DIM), lambda i: (0, 0)),
        ],
        out_specs=pl.BlockSpec((blk, WIDE), lambda i: (i, 0)),
        out_shape=jax.ShapeDtypeStruct((VOCAB, WIDE), jnp.float32),
    )(table, W)


def _tree_sum(vals):
    while len(vals) > 1:
        nxt = [a + b for a, b in zip(vals[0::2], vals[1::2])]
        if len(vals) % 2:
            nxt.append(vals[-1])
        vals = nxt
    return vals[0]


def _sc_pool(tok2, table_w):
    mesh = plsc.VectorSubcoreMesh(core_axis_name="c", subcore_axis_name="s")

    @functools.partial(
        pl.kernel,
        mesh=mesh,
        out_type=jax.ShapeDtypeStruct((BATCH // 2, WIDE), jnp.float32),
        scratch_types=[
            pltpu.VMEM((_TOK_W // WIDE, WIDE), jnp.int32),
            pltpu.VMEM((_NGRP, 80), jnp.int32),
            pltpu.VMEM((2, _G, 80, WIDE), jnp.float32),
            pltpu.VMEM((_PER_W // 2, WIDE), jnp.float32),
            pltpu.SemaphoreType.DMA,
            pltpu.SemaphoreType.DMA,
        ],
    )
    def k(tok_hbm, tw_hbm, out_hbm, stage_v, idx_v, rows_v, pooled_v,
          sem0, sem1):
        wid = lax.axis_index("s") * _NC + lax.axis_index("c")
        sems = (sem0, sem1)

        pltpu.sync_copy(
            tok_hbm.at[pl.ds(wid * (_TOK_W // WIDE), _TOK_W // WIDE), :],
            stage_v)

        def repart_body(r, carry):
            for kk in range(5):
                pos = r * 80 + kk * 16
                idx_v[r, pl.ds(kk * 16, 16)] = (
                    stage_v[pos // WIDE, pl.ds(pos % WIDE, 16)])
            return carry

        lax.fori_loop(0, _NGRP, repart_body, 0)

        def fire(c, p):
            for g in range(_G):
                pltpu.async_copy(
                    tw_hbm.at[idx_v.at[c * _G + g]], rows_v.at[p, g], sems[p])

        def drain(c, p):
            for g in range(_G):
                pltpu.make_async_copy(
                    tw_hbm.at[idx_v.at[c * _G + g]], rows_v.at[p, g],
                    sems[p]).wait()

        fire(0, 0)
        fire(1, 1)

        def pair_body(h, carry):
            for p in range(2):
                c = h * 2 + p
                drain(c, p)
                for i in range(_C):
                    g, r0 = i // 4, (i % 4) * 20
                    col = (i % 2) * DIM
                    for j in range(DIM // 16):
                        acc = _tree_sum([
                            rows_v[p, g, r0 + t, pl.ds(j * 16, 16)]
                            for t in range(POOL)])
                        pooled_v[c * (_C // 2) + i // 2,
                                 pl.ds(col + j * 16, 16)] = acc

                @pl.when(c + 2 < _NCHUNK)
                def _():
                    fire(c + 2, p)

            return carry

        lax.fori_loop(0, _NCHUNK // 2, pair_body, 0)
        pltpu.sync_copy(pooled_v,
                        out_hbm.at[pl.ds(wid * (_PER_W // 2), _PER_W // 2), :])

    return k(tok2, table_w)


def kernel(document, table, W):
    tok2 = document[:, :POOL].astype(jnp.int32).reshape(
        BATCH * POOL // WIDE, WIDE)
    table_w = _tc_table_w(table, W)
    out2 = _sc_pool(tok2, table_w)
    return out2.reshape(BATCH, DIM)


# speedup vs baseline: 1.0914x; 1.0914x over previous
import functools

import jax
import jax.numpy as jnp
from jax import lax
from jax.experimental import pallas as pl
from jax.experimental.pallas import tpu as pltpu
from jax.experimental.pallas import tpu_sc as plsc

BATCH = 16384
SEQ = 50
POOL = 20
VOCAB = 1000000
DIM = 64
WIDE = 128

_INFO = plsc.get_sparse_core_info()
_NC = _INFO.num_cores
_NS = _INFO.num_subcores
_NW = _NC * _NS
_PER_W = BATCH // _NW
_TOK_W = _PER_W * POOL
_C = 8
_G = 2
_NCHUNK = _PER_W // _C
_NGRP = _TOK_W // 80


def _tc_table_w(table, W):
    blk = 8000
    sub = blk // 4

    Wd = jnp.zeros((4 * DIM, 4 * DIM), jnp.float32)
    for j in range(4):
        Wd = Wd.at[j * DIM:(j + 1) * DIM, j * DIM:(j + 1) * DIM].set(W.T)

    def mm(x0, x1, x2, x3, w_ref, o_ref):
        x4 = jnp.concatenate([x0[...], x1[...], x2[...], x3[...]], axis=1)
        o = lax.dot_general(
            x4, w_ref[...],
            dimension_numbers=(((1,), (0,)), ((), ())),
            precision=lax.Precision.HIGHEST,
            preferred_element_type=jnp.float32,
        )
        v = jnp.concatenate([o[:, j * DIM:(j + 1) * DIM] for j in range(4)],
                            axis=0)
        o_ref[...] = jnp.concatenate([v, jnp.zeros_like(v)], axis=1)

    return pl.pallas_call(
        mm,
        grid=(VOCAB // blk,),
        in_specs=[
            pl.BlockSpec((sub, DIM), lambda i, j=j: (4 * i + j, 0))
            for j in range(4)
        ] + [pl.BlockSpec((4 * DIM, 4 * DIM), lambda i: (0, 0))],
        out_specs=pl.BlockSpec((blk, WIDE), lambda i: (i, 0)),
        out_shape=jax.ShapeDtypeStruct((VOCAB, WIDE), jnp.float32),
    )(table, table, table, table, Wd)


def _tree_sum(vals):
    while len(vals) > 1:
        nxt = [a + b for a, b in zip(vals[0::2], vals[1::2])]
        if len(vals) % 2:
            nxt.append(vals[-1])
        vals = nxt
    return vals[0]


def _sc_pool(tok2, table_w):
    mesh = plsc.VectorSubcoreMesh(core_axis_name="c", subcore_axis_name="s")

    @functools.partial(
        pl.kernel,
        mesh=mesh,
        out_type=jax.ShapeDtypeStruct((BATCH // 2, WIDE), jnp.float32),
        scratch_types=[
            pltpu.VMEM((_TOK_W // WIDE, WIDE), jnp.int32),
            pltpu.VMEM((_NGRP, 80), jnp.int32),
            pltpu.VMEM((2, _G, 80, WIDE), jnp.float32),
            pltpu.VMEM((_PER_W // 2, WIDE), jnp.float32),
            pltpu.SemaphoreType.DMA,
            pltpu.SemaphoreType.DMA,
        ],
    )
    def k(tok_hbm, tw_hbm, out_hbm, stage_v, idx_v, rows_v, pooled_v,
          sem0, sem1):
        wid = lax.axis_index("s") * _NC + lax.axis_index("c")
        sems = (sem0, sem1)

        pltpu.sync_copy(
            tok_hbm.at[pl.ds(wid * (_TOK_W // WIDE), _TOK_W // WIDE), :],
            stage_v)

        def repart_body(r, carry):
            for kk in range(5):
                pos = r * 80 + kk * 16
                idx_v[r, pl.ds(kk * 16, 16)] = (
                    stage_v[pos // WIDE, pl.ds(pos % WIDE, 16)])
            return carry

        lax.fori_loop(0, _NGRP, repart_body, 0)

        def fire(c, p):
            for g in range(_G):
                pltpu.async_copy(
                    tw_hbm.at[idx_v.at[c * _G + g]], rows_v.at[p, g], sems[p])

        def drain(c, p):
            for g in range(_G):
                pltpu.make_async_copy(
                    tw_hbm.at[idx_v.at[c * _G + g]], rows_v.at[p, g],
                    sems[p]).wait()

        fire(0, 0)
        fire(1, 1)

        def pair_body(h, carry):
            for p in range(2):
                c = h * 2 + p
                drain(c, p)
                for i in range(_C):
                    g, r0 = i // 4, (i % 4) * 20
                    col = (i % 2) * DIM
                    for j in range(DIM // 16):
                        acc = _tree_sum([
                            rows_v[p, g, r0 + t, pl.ds(j * 16, 16)]
                            for t in range(POOL)])
                        pooled_v[c * (_C // 2) + i // 2,
                                 pl.ds(col + j * 16, 16)] = acc

                @pl.when(c + 2 < _NCHUNK)
                def _():
                    fire(c + 2, p)

            return carry

        lax.fori_loop(0, _NCHUNK // 2, pair_body, 0)
        pltpu.sync_copy(pooled_v,
                        out_hbm.at[pl.ds(wid * (_PER_W // 2), _PER_W // 2), :])

    return k(tok2, table_w)


def kernel(document, table, W):
    tok2 = document[:, :POOL].astype(jnp.int32).reshape(
        BATCH * POOL // WIDE, WIDE)
    table_w = _tc_table_w(table, W)
    out2 = _sc_pool(tok2, table_w)
    return out2.reshape(BATCH, DIM)


# speedup vs baseline: 1.0932x; 1.0016x over previous
import functools

import jax
import jax.numpy as jnp
from jax import lax
from jax.experimental import pallas as pl
from jax.experimental.pallas import tpu as pltpu
from jax.experimental.pallas import tpu_sc as plsc

BATCH = 16384
SEQ = 50
POOL = 20
VOCAB = 1000000
DIM = 64
WIDE = 128

_INFO = plsc.get_sparse_core_info()
_NC = _INFO.num_cores
_NS = _INFO.num_subcores
_NW = _NC * _NS
_PER_W = BATCH // _NW
_TOK_W = _PER_W * POOL
_C = 8
_G = 2
_NCHUNK = _PER_W // _C
_NGRP = _TOK_W // 80


def _tc_table_w(table, W):
    blk = 8000
    sub = blk // 4

    Wd = jnp.zeros((4 * DIM, 4 * DIM), jnp.float32)
    for j in range(4):
        Wd = Wd.at[j * DIM:(j + 1) * DIM, j * DIM:(j + 1) * DIM].set(W.T)

    def mm(x0, x1, x2, x3, w_ref, o_ref):
        x4 = jnp.concatenate([x0[...], x1[...], x2[...], x3[...]], axis=1)
        o = lax.dot_general(
            x4, w_ref[...],
            dimension_numbers=(((1,), (0,)), ((), ())),
            precision=lax.Precision.HIGHEST,
            preferred_element_type=jnp.float32,
        )
        v = jnp.concatenate([o[:, j * DIM:(j + 1) * DIM] for j in range(4)],
                            axis=0)
        o_ref[...] = jnp.concatenate([v, jnp.zeros_like(v)], axis=1)

    return pl.pallas_call(
        mm,
        grid=(VOCAB // blk,),
        in_specs=[
            pl.BlockSpec((sub, DIM), lambda i, j=j: (4 * i + j, 0))
            for j in range(4)
        ] + [pl.BlockSpec((4 * DIM, 4 * DIM), lambda i: (0, 0))],
        out_specs=pl.BlockSpec((blk, WIDE), lambda i: (i, 0)),
        out_shape=jax.ShapeDtypeStruct((VOCAB, WIDE), jnp.float32),
    )(table, table, table, table, Wd)


def _tree_sum(vals):
    while len(vals) > 1:
        nxt = [a + b for a, b in zip(vals[0::2], vals[1::2])]
        if len(vals) % 2:
            nxt.append(vals[-1])
        vals = nxt
    return vals[0]


def _sc_pool(tok2, table_w):
    mesh = plsc.VectorSubcoreMesh(core_axis_name="c", subcore_axis_name="s")

    @functools.partial(
        pl.kernel,
        mesh=mesh,
        out_type=jax.ShapeDtypeStruct((BATCH // 2, WIDE), jnp.float32),
        compiler_params=pltpu.CompilerParams(use_tc_tiling_on_sc=False),
        scratch_types=[
            pltpu.VMEM((_TOK_W // WIDE, WIDE), jnp.int32),
            pltpu.VMEM((_NGRP, 80), jnp.int32),
            pltpu.VMEM((2, _G, 80, WIDE), jnp.float32),
            pltpu.VMEM((_PER_W // 2, WIDE), jnp.float32),
            pltpu.SemaphoreType.DMA,
            pltpu.SemaphoreType.DMA,
        ],
    )
    def k(tok_hbm, tw_hbm, out_hbm, stage_v, idx_v, rows_v, pooled_v,
          sem0, sem1):
        wid = lax.axis_index("s") * _NC + lax.axis_index("c")
        sems = (sem0, sem1)

        pltpu.sync_copy(
            tok_hbm.at[pl.ds(wid * (_TOK_W // WIDE), _TOK_W // WIDE), :],
            stage_v)

        def repart_body(r, carry):
            for kk in range(5):
                pos = r * 80 + kk * 16
                idx_v[r, pl.ds(kk * 16, 16)] = (
                    stage_v[pos // WIDE, pl.ds(pos % WIDE, 16)])
            return carry

        lax.fori_loop(0, _NGRP, repart_body, 0)

        def fire(c, p):
            for g in range(_G):
                pltpu.async_copy(
                    tw_hbm.at[idx_v.at[c * _G + g]], rows_v.at[p, g], sems[p])

        def drain(c, p):
            for g in range(_G):
                pltpu.make_async_copy(
                    tw_hbm.at[idx_v.at[c * _G + g]], rows_v.at[p, g],
                    sems[p]).wait()

        fire(0, 0)
        fire(1, 1)

        def pair_body(h, carry):
            for p in range(2):
                c = h * 2 + p
                drain(c, p)
                for i in range(_C):
                    g, r0 = i // 4, (i % 4) * 20
                    col = (i % 2) * DIM
                    for j in range(DIM // 16):
                        acc = _tree_sum([
                            rows_v[p, g, r0 + t, pl.ds(j * 16, 16)]
                            for t in range(POOL)])
                        pooled_v[c * (_C // 2) + i // 2,
                                 pl.ds(col + j * 16, 16)] = acc

                @pl.when(c + 2 < _NCHUNK)
                def _():
                    fire(c + 2, p)

            return carry

        lax.fori_loop(0, _NCHUNK // 2, pair_body, 0)
        pltpu.sync_copy(pooled_v,
                        out_hbm.at[pl.ds(wid * (_PER_W // 2), _PER_W // 2), :])

    return k(tok2, table_w)


def kernel(document, table, W):
    tok2 = document[:, :POOL].astype(jnp.int32).reshape(
        BATCH * POOL // WIDE, WIDE)
    table_w = _tc_table_w(table, W)
    out2 = _sc_pool(tok2, table_w)
    return out2.reshape(BATCH, DIM)


# speedup vs baseline: 1.9795x; 1.8107x over previous
import functools

import jax
import jax.numpy as jnp
from jax import lax
from jax.experimental import pallas as pl
from jax.experimental.pallas import tpu as pltpu
from jax.experimental.pallas import tpu_sc as plsc

BATCH = 16384
SEQ = 50
POOL = 20
VOCAB = 1000000
DIM = 64
WIDE = 128

_INFO = plsc.get_sparse_core_info()
_NC = _INFO.num_cores
_NS = _INFO.num_subcores
_NW = _NC * _NS
_PER_W = BATCH // _NW
_TOK_W = _PER_W * POOL
_C = 8
_G = 2
_NCHUNK = _PER_W // _C
_NGRP = _TOK_W // 80


def _tc_table_w(table, W):
    sub = 2048
    blk = 4 * sub
    main = (VOCAB // blk) * blk
    tail = VOCAB - main

    Wd = jnp.zeros((4 * DIM, 4 * DIM), jnp.float32)
    for j in range(4):
        Wd = Wd.at[j * DIM:(j + 1) * DIM, j * DIM:(j + 1) * DIM].set(W)

    def mm(x0, x1, x2, x3, w_ref, o_ref):
        x4 = jnp.concatenate([x0[...], x1[...], x2[...], x3[...]], axis=0)
        o = lax.dot_general(
            x4, w_ref[...],
            dimension_numbers=(((0,), (1,)), ((), ())),
            precision=lax.Precision.HIGHEST,
            preferred_element_type=jnp.float32,
        )
        v = jnp.concatenate([o[:, j * DIM:(j + 1) * DIM] for j in range(4)],
                            axis=0)
        o_ref[...] = jnp.concatenate([v, jnp.zeros_like(v)], axis=1)

    tw = pl.pallas_call(
        mm,
        grid=(main // blk,),
        in_specs=[
            pl.BlockSpec((DIM, sub), lambda i, j=j: (0, 4 * i + j))
            for j in range(4)
        ] + [pl.BlockSpec((4 * DIM, 4 * DIM), lambda i: (0, 0))],
        out_specs=pl.BlockSpec((blk, WIDE), lambda i: (i, 0)),
        out_shape=jax.ShapeDtypeStruct((VOCAB, WIDE), jnp.float32),
    )(table.T, table.T, table.T, table.T, Wd)

    def mm_tail(x_ref, w_ref, o_ref):
        o = lax.dot_general(
            x_ref[...], w_ref[...],
            dimension_numbers=(((0,), (1,)), ((), ())),
            precision=lax.Precision.HIGHEST,
            preferred_element_type=jnp.float32,
        )
        o_ref[...] = jnp.concatenate([o, jnp.zeros_like(o)], axis=1)

    tw_tail = pl.pallas_call(
        mm_tail,
        out_shape=jax.ShapeDtypeStruct((tail, WIDE), jnp.float32),
    )(lax.slice(table.T, (0, main), (DIM, VOCAB)), W)

    return lax.dynamic_update_slice(tw, tw_tail, (main, 0))


def _tree_sum(vals):
    while len(vals) > 1:
        nxt = [a + b for a, b in zip(vals[0::2], vals[1::2])]
        if len(vals) % 2:
            nxt.append(vals[-1])
        vals = nxt
    return vals[0]


def _sc_pool(tok2, table_w):
    mesh = plsc.VectorSubcoreMesh(core_axis_name="c", subcore_axis_name="s")

    @functools.partial(
        pl.kernel,
        mesh=mesh,
        out_type=jax.ShapeDtypeStruct((BATCH // 2, WIDE), jnp.float32),
        compiler_params=pltpu.CompilerParams(use_tc_tiling_on_sc=False),
        scratch_types=[
            pltpu.VMEM((_TOK_W // WIDE, WIDE), jnp.int32),
            pltpu.VMEM((_NGRP, 80), jnp.int32),
            pltpu.VMEM((2, _G, 80, WIDE), jnp.float32),
            pltpu.VMEM((_PER_W // 2, WIDE), jnp.float32),
            pltpu.SemaphoreType.DMA,
            pltpu.SemaphoreType.DMA,
        ],
    )
    def k(tok_hbm, tw_hbm, out_hbm, stage_v, idx_v, rows_v, pooled_v,
          sem0, sem1):
        wid = lax.axis_index("s") * _NC + lax.axis_index("c")
        sems = (sem0, sem1)

        pltpu.sync_copy(
            tok_hbm.at[pl.ds(wid * (_TOK_W // WIDE), _TOK_W // WIDE), :],
            stage_v)

        def repart_body(r, carry):
            for kk in range(5):
                pos = r * 80 + kk * 16
                idx_v[r, pl.ds(kk * 16, 16)] = (
                    stage_v[pos // WIDE, pl.ds(pos % WIDE, 16)])
            return carry

        lax.fori_loop(0, _NGRP, repart_body, 0)

        def fire(c, p):
            for g in range(_G):
                pltpu.async_copy(
                    tw_hbm.at[idx_v.at[c * _G + g]], rows_v.at[p, g], sems[p])

        def drain(c, p):
            for g in range(_G):
                pltpu.make_async_copy(
                    tw_hbm.at[idx_v.at[c * _G + g]], rows_v.at[p, g],
                    sems[p]).wait()

        fire(0, 0)
        fire(1, 1)

        def pair_body(h, carry):
            for p in range(2):
                c = h * 2 + p
                drain(c, p)
                for i in range(_C):
                    g, r0 = i // 4, (i % 4) * 20
                    col = (i % 2) * DIM
                    for j in range(DIM // 16):
                        acc = _tree_sum([
                            rows_v[p, g, r0 + t, pl.ds(j * 16, 16)]
                            for t in range(POOL)])
                        pooled_v[c * (_C // 2) + i // 2,
                                 pl.ds(col + j * 16, 16)] = acc

                @pl.when(c + 2 < _NCHUNK)
                def _():
                    fire(c + 2, p)

            return carry

        lax.fori_loop(0, _NCHUNK // 2, pair_body, 0)
        pltpu.sync_copy(pooled_v,
                        out_hbm.at[pl.ds(wid * (_PER_W // 2), _PER_W // 2), :])

    return k(tok2, table_w)


def kernel(document, table, W):
    tok2 = document[:, :POOL].astype(jnp.int32).reshape(
        BATCH * POOL // WIDE, WIDE)
    table_w = _tc_table_w(table, W)
    out2 = _sc_pool(tok2, table_w)
    return out2.reshape(BATCH, DIM)
